# baseline (speedup 1.0000x reference)
import jax
import jax.numpy as jnp
from jax import lax
from jax.experimental import pallas as pl
from jax.experimental.pallas import tpu as pltpu
from jax.experimental.pallas import tpu_sc as plsc

N = 10000
CH = 128
E = 320000
NC = 2
NS = 16
NW = NC * NS
CHUNK = 128
NBUF = 2
NPH = 2
HCH = 40
CHUNKS = NPH * HCH
E_PAD = NW * CHUNKS * CHUNK
ACC_ROWS = 10016
ZROWS = ACC_ROWS // NS
OROWS = 624


def _agg_body(h_hbm, srcw_hbm, dstw_hbm, out_hbm,
              src_idx, dst_idx, rows0, rows1,
              sem0, sem1, gsh):
    c = lax.axis_index("c")
    s = lax.axis_index("s")
    wid = c * NS + s
    rows = (rows0, rows1)
    sems = (sem0, sem1)

    pltpu.sync_copy(srcw_hbm.at[wid, pl.ds(0, HCH)], src_idx)
    pltpu.sync_copy(dstw_hbm.at[wid, pl.ds(0, HCH)], dst_idx)
    pltpu.async_copy(h_hbm.at[src_idx.at[0]], rows0, sem0)

    zero16 = jnp.zeros((16,), jnp.float32)

    def zrow(i, carry):
        for k in range(8):
            rows1[i, pl.ds(k * 16, 16)] = zero16
        return carry

    lax.fori_loop(0, CHUNK, zrow, 0)
    zbase = s * ZROWS
    nfull = ZROWS // CHUNK
    for p in range(nfull):
        pltpu.sync_copy(rows1, gsh.at[pl.ds(zbase + p * CHUNK, CHUNK)])
    zrem = ZROWS - nfull * CHUNK
    if zrem:
        pltpu.sync_copy(rows1.at[pl.ds(0, zrem)],
                        gsh.at[pl.ds(zbase + nfull * CHUNK, zrem)])
    plsc.subcore_barrier()

    for ph in range(NPH):
        if ph:
            pltpu.sync_copy(srcw_hbm.at[wid, pl.ds(ph * HCH, HCH)], src_idx)
            pltpu.sync_copy(dstw_hbm.at[wid, pl.ds(ph * HCH, HCH)], dst_idx)
        for b in range(NBUF):
            if ph == 0 and b == 0:
                continue
            pltpu.async_copy(h_hbm.at[src_idx.at[b]], rows[b], sems[b])

        def group(g, carry):
            for b in range(NBUF):
                j = NBUF * g + b
                pltpu.make_async_copy(h_hbm.at[src_idx.at[j]],
                                      rows[b], sems[b]).wait()
                pltpu.sync_copy(rows[b], gsh.at[dst_idx.at[j]], add=True)
                pltpu.async_copy(h_hbm.at[src_idx.at[j + NBUF]],
                                 rows[b], sems[b])
            return carry

        lax.fori_loop(0, HCH // NBUF - 1, group, 0)
        for b in range(NBUF):
            j = HCH - NBUF + b
            pltpu.make_async_copy(h_hbm.at[src_idx.at[j]],
                                  rows[b], sems[b]).wait()
            pltpu.sync_copy(rows[b], gsh.at[dst_idx.at[j]], add=True)
    plsc.subcore_barrier()

    pltpu.sync_copy(gsh.at[pl.ds(s * OROWS, OROWS)],
                    out_hbm.at[c, pl.ds(s * OROWS, OROWS)])
    tail = NS * OROWS

    @pl.when(s == NS - 1)
    def _():
        pltpu.sync_copy(gsh.at[pl.ds(tail, N - tail)],
                        out_hbm.at[c, pl.ds(tail, N - tail)])


_agg = pl.kernel(
    _agg_body,
    out_type=jax.ShapeDtypeStruct((NC, N, CH), jnp.float32),
    mesh=plsc.VectorSubcoreMesh(core_axis_name="c", subcore_axis_name="s"),
    scratch_types=[
        pltpu.VMEM((HCH, CHUNK), jnp.int32),
        pltpu.VMEM((HCH, CHUNK), jnp.int32),
        pltpu.VMEM((CHUNK, CH), jnp.float32),
        pltpu.VMEM((CHUNK, CH), jnp.float32),
        pltpu.SemaphoreType.DMA,
        pltpu.SemaphoreType.DMA,
        pltpu.VMEM_SHARED((ACC_ROWS, CH), jnp.float32),
    ],
)


_BLK = 10000


def _self_body(h_ref, wself_ref, b_ref, o_ref):
    o_ref[...] = jnp.dot(h_ref[...], wself_ref[...],
                         preferred_element_type=jnp.float32) + b_ref[...]


_mm_self = pl.pallas_call(
    _self_body,
    grid=(N // _BLK,),
    in_specs=[
        pl.BlockSpec((_BLK, CH), lambda i: (i, 0)),
        pl.BlockSpec((CH, CH), lambda i: (0, 0)),
        pl.BlockSpec((1, CH), lambda i: (0, 0)),
    ],
    out_specs=pl.BlockSpec((_BLK, CH), lambda i: (i, 0)),
    out_shape=jax.ShapeDtypeStruct((N, CH), jnp.float32),
)


def _fin_body(g2_ref, s_ref, wnb_ref, o_ref):
    g = g2_ref[0] + g2_ref[1]
    y = jnp.dot(g, wnb_ref[...], preferred_element_type=jnp.float32)
    o_ref[...] = jnp.maximum(y + s_ref[...], 0.0)


_mm_fin = pl.pallas_call(
    _fin_body,
    grid=(N // _BLK,),
    in_specs=[
        pl.BlockSpec((NC, _BLK, CH), lambda i: (0, i, 0)),
        pl.BlockSpec((_BLK, CH), lambda i: (i, 0)),
        pl.BlockSpec((CH, CH), lambda i: (0, 0)),
    ],
    out_specs=pl.BlockSpec((_BLK, CH), lambda i: (i, 0)),
    out_shape=jax.ShapeDtypeStruct((N, CH), jnp.float32),
)


def kernel(x, edge_index, Win_nb, Win_self, b_in,
           Wc_nb, Wc_self, bc, W2_nb, W2_self, b2):
    src = edge_index[0].astype(jnp.int32)
    dst = edge_index[1].astype(jnp.int32)
    npad = E_PAD - E
    pad_src = (jnp.arange(npad, dtype=jnp.int32) * 97) % N
    pad_dst = N + jnp.arange(npad, dtype=jnp.int32) % (ACC_ROWS - N)
    srcw = jnp.concatenate([src, pad_src]).reshape(NW, CHUNKS, CHUNK)
    dstw = jnp.concatenate([dst, pad_dst]).reshape(NW, CHUNKS, CHUNK)

    layers = [(Win_nb, Win_self, b_in)]
    for i in range(Wc_nb.shape[0]):
        layers.append((Wc_nb[i], Wc_self[i], bc[i]))
        layers.append((W2_nb[i], W2_self[i], b2[i]))

    h = x
    for (wnb, wself, b) in layers:
        g2 = _agg(h, srcw, dstw)
        s = _mm_self(h, wself, b.reshape(1, CH))
        h = _mm_fin(g2, s, wnb)
    return h


# speedup vs baseline: 1.0080x; 1.0080x over previous
import jax
import jax.numpy as jnp
from jax import lax
from jax.experimental import pallas as pl
from jax.experimental.pallas import tpu as pltpu
from jax.experimental.pallas import tpu_sc as plsc

N = 10000
CH = 128
E = 320000
NC = 2
NS = 16
NW = NC * NS
CHUNK = 128
NBUF = 2
NPH = 2
HCH = 40
CHUNKS = NPH * HCH
E_PAD = NW * CHUNKS * CHUNK
ACC_ROWS = 10016
ZROWS = ACC_ROWS // NS
OROWS = 624


def _agg_body(h_hbm, srcw_hbm, dstw_hbm, out_hbm,
              src_idx, dst_idx, rows0, rows1,
              sem0, sem1, gsh):
    c = lax.axis_index("c")
    s = lax.axis_index("s")
    wid = c * NS + s
    rows = (rows0, rows1)
    sems = (sem0, sem1)

    pltpu.sync_copy(srcw_hbm.at[wid, pl.ds(0, HCH)], src_idx)
    pltpu.sync_copy(dstw_hbm.at[wid, pl.ds(0, HCH)], dst_idx)
    pltpu.async_copy(h_hbm.at[src_idx.at[0]], rows0, sem0)

    zero16 = jnp.zeros((16,), jnp.float32)

    def zrow(i, carry):
        for k in range(8):
            rows1[i, pl.ds(k * 16, 16)] = zero16
        return carry

    lax.fori_loop(0, CHUNK, zrow, 0)
    zbase = s * ZROWS
    nfull = ZROWS // CHUNK
    for p in range(nfull):
        pltpu.sync_copy(rows1, gsh.at[pl.ds(zbase + p * CHUNK, CHUNK)])
    zrem = ZROWS - nfull * CHUNK
    if zrem:
        pltpu.sync_copy(rows1.at[pl.ds(0, zrem)],
                        gsh.at[pl.ds(zbase + nfull * CHUNK, zrem)])
    plsc.subcore_barrier()

    for ph in range(NPH):
        if ph:
            pltpu.sync_copy(srcw_hbm.at[wid, pl.ds(ph * HCH, HCH)], src_idx)
            pltpu.sync_copy(dstw_hbm.at[wid, pl.ds(ph * HCH, HCH)], dst_idx)
        for b in range(NBUF):
            if ph == 0 and b == 0:
                continue
            pltpu.async_copy(h_hbm.at[src_idx.at[b]], rows[b], sems[b])

        def group(g, carry):
            for b in range(NBUF):
                j = NBUF * g + b
                pltpu.make_async_copy(h_hbm.at[src_idx.at[j]],
                                      rows[b], sems[b]).wait()
                pltpu.sync_copy(rows[b], gsh.at[dst_idx.at[j]], add=True)
                pltpu.async_copy(h_hbm.at[src_idx.at[j + NBUF]],
                                 rows[b], sems[b])
            return carry

        lax.fori_loop(0, HCH // NBUF - 1, group, 0)
        for b in range(NBUF):
            j = HCH - NBUF + b
            pltpu.make_async_copy(h_hbm.at[src_idx.at[j]],
                                  rows[b], sems[b]).wait()
            pltpu.sync_copy(rows[b], gsh.at[dst_idx.at[j]], add=True)
    plsc.subcore_barrier()

    pltpu.sync_copy(gsh.at[pl.ds(s * OROWS, OROWS)],
                    out_hbm.at[c, pl.ds(s * OROWS, OROWS)])
    tail = NS * OROWS

    @pl.when(s == NS - 1)
    def _():
        pltpu.sync_copy(gsh.at[pl.ds(tail, N - tail)],
                        out_hbm.at[c, pl.ds(tail, N - tail)])


_agg = pl.kernel(
    _agg_body,
    out_type=jax.ShapeDtypeStruct((NC, N, CH), jnp.float32),
    mesh=plsc.VectorSubcoreMesh(core_axis_name="c", subcore_axis_name="s"),
    scratch_types=[
        pltpu.VMEM((HCH, CHUNK), jnp.int32),
        pltpu.VMEM((HCH, CHUNK), jnp.int32),
        pltpu.VMEM((CHUNK, CH), jnp.float32),
        pltpu.VMEM((CHUNK, CH), jnp.float32),
        pltpu.SemaphoreType.DMA,
        pltpu.SemaphoreType.DMA,
        pltpu.VMEM_SHARED((ACC_ROWS, CH), jnp.float32),
    ],
)


_BLK = 5000


def _self_body(h_ref, wself_ref, b_ref, o_ref):
    o_ref[...] = jnp.dot(h_ref[...], wself_ref[...],
                         preferred_element_type=jnp.float32) + b_ref[...]


_mm_self = pl.pallas_call(
    _self_body,
    grid=(N // _BLK,),
    in_specs=[
        pl.BlockSpec((_BLK, CH), lambda i: (i, 0)),
        pl.BlockSpec((CH, CH), lambda i: (0, 0)),
        pl.BlockSpec((1, CH), lambda i: (0, 0)),
    ],
    out_specs=pl.BlockSpec((_BLK, CH), lambda i: (i, 0)),
    out_shape=jax.ShapeDtypeStruct((N, CH), jnp.float32),
)


def _all_body(g2_ref, h_ref, wnb_ref, wself_ref, b_ref, o_ref):
    g = g2_ref[0] + g2_ref[1]
    y = jnp.dot(g, wnb_ref[...], preferred_element_type=jnp.float32)
    y = y + jnp.dot(h_ref[...], wself_ref[...],
                    preferred_element_type=jnp.float32)
    o_ref[...] = jnp.maximum(y + b_ref[...], 0.0)


_mm_all = pl.pallas_call(
    _all_body,
    grid=(N // _BLK,),
    in_specs=[
        pl.BlockSpec((NC, _BLK, CH), lambda i: (0, i, 0)),
        pl.BlockSpec((_BLK, CH), lambda i: (i, 0)),
        pl.BlockSpec((CH, CH), lambda i: (0, 0)),
        pl.BlockSpec((CH, CH), lambda i: (0, 0)),
        pl.BlockSpec((1, CH), lambda i: (0, 0)),
    ],
    out_specs=pl.BlockSpec((_BLK, CH), lambda i: (i, 0)),
    out_shape=jax.ShapeDtypeStruct((N, CH), jnp.float32),
)


def _fin_body(g2_ref, s_ref, wnb_ref, o_ref):
    g = g2_ref[0] + g2_ref[1]
    y = jnp.dot(g, wnb_ref[...], preferred_element_type=jnp.float32)
    o_ref[...] = jnp.maximum(y + s_ref[...], 0.0)


_mm_fin = pl.pallas_call(
    _fin_body,
    grid=(N // _BLK,),
    in_specs=[
        pl.BlockSpec((NC, _BLK, CH), lambda i: (0, i, 0)),
        pl.BlockSpec((_BLK, CH), lambda i: (i, 0)),
        pl.BlockSpec((CH, CH), lambda i: (0, 0)),
    ],
    out_specs=pl.BlockSpec((_BLK, CH), lambda i: (i, 0)),
    out_shape=jax.ShapeDtypeStruct((N, CH), jnp.float32),
)


def kernel(x, edge_index, Win_nb, Win_self, b_in,
           Wc_nb, Wc_self, bc, W2_nb, W2_self, b2):
    src = edge_index[0].astype(jnp.int32)
    dst = edge_index[1].astype(jnp.int32)
    npad = E_PAD - E
    pad_src = (jnp.arange(npad, dtype=jnp.int32) * 97) % N
    pad_dst = N + jnp.arange(npad, dtype=jnp.int32) % (ACC_ROWS - N)
    srcw = jnp.concatenate([src, pad_src]).reshape(NW, CHUNKS, CHUNK)
    dstw = jnp.concatenate([dst, pad_dst]).reshape(NW, CHUNKS, CHUNK)

    layers = [(Win_nb, Win_self, b_in)]
    for i in range(Wc_nb.shape[0]):
        layers.append((Wc_nb[i], Wc_self[i], bc[i]))
        layers.append((W2_nb[i], W2_self[i], b2[i]))

    h = x
    for (wnb, wself, b) in layers:
        g2 = _agg(h, srcw, dstw)
        h = _mm_all(g2, h, wnb, wself, b.reshape(1, CH))
    return h
